# Initial kernel scaffold; baseline (speedup 1.0000x reference)
import functools

import jax
import jax.numpy as jnp
from jax.experimental import pallas as pl


def _conv2d(x, w, b, stride, pad):
    out = jax.lax.conv_general_dilated(
        x, w, window_strides=(stride, stride),
        padding=[(pad, pad), (pad, pad)],
        dimension_numbers=('NCHW', 'OIHW', 'NCHW'))
    return out + b[None, :, None, None]


def _deconv2d(x, w, b, stride, pad):
    w_f = jnp.flip(w, axis=(2, 3)).transpose(1, 0, 2, 3)
    k = w.shape[2]
    p = k - 1 - pad
    out = jax.lax.conv_general_dilated(
        x, w_f, window_strides=(1, 1), padding=[(p, p), (p, p)],
        lhs_dilation=(stride, stride),
        dimension_numbers=('NCHW', 'OIHW', 'NCHW'))
    return out + b[None, :, None, None]


def _vq_kernel(flat_ref, emb_ref, q_ref, sq_ref):
    i = pl.program_id(0)
    f = flat_ref[...]
    e = emb_ref[...]
    fe = jax.lax.dot_general(f, e, (((1,), (1,)), ((), ())),
                             preferred_element_type=jnp.float32)
    d2 = (jnp.sum(f * f, axis=1, keepdims=True) - 2.0 * fe
          + jnp.sum(e * e, axis=1)[None, :])
    dist = jnp.sqrt(jnp.maximum(d2, 0.0))
    idx = jnp.argmin(dist, axis=1)
    onehot = (idx[:, None] ==
              jax.lax.broadcasted_iota(jnp.int32, dist.shape, 1)
              ).astype(jnp.float32)
    q = jax.lax.dot_general(onehot, e, (((1,), (0,)), ((), ())),
                            preferred_element_type=jnp.float32)
    q_ref[...] = q
    diff = q - f
    part = jnp.sum(diff * diff)

    @pl.when(i == 0)
    def _init():
        sq_ref[0, 0] = part

    @pl.when(i != 0)
    def _acc():
        sq_ref[0, 0] += part


@functools.partial(jax.jit, static_argnames=("blk",))
def _vq(flat, emb, blk=512):
    n, c = flat.shape
    k = emb.shape[0]
    grid = n // blk
    q, sq = pl.pallas_call(
        _vq_kernel,
        grid=(grid,),
        in_specs=[
            pl.BlockSpec((blk, c), lambda i: (i, 0)),
            pl.BlockSpec((k, c), lambda i: (0, 0)),
        ],
        out_specs=[
            pl.BlockSpec((blk, c), lambda i: (i, 0)),
            pl.BlockSpec((1, 1), lambda i: (0, 0)),
        ],
        out_shape=[
            jax.ShapeDtypeStruct((n, c), jnp.float32),
            jax.ShapeDtypeStruct((1, 1), jnp.float32),
        ],
    )(flat, emb)
    return q, sq[0, 0]


def kernel(x, w1, b1, w2, b2, emb, dw1, db1, dw2, db2, w3, b3):
    z = jax.nn.relu(_conv2d(x, w1, b1, 2, 1))
    z = jax.nn.relu(_conv2d(z, w2, b2, 2, 1))
    B, C, H, W = z.shape
    flat = z.transpose(0, 2, 3, 1).reshape(-1, C)
    q_flat, sq = _vq(flat, emb)
    loss = 1.25 * sq / (flat.shape[0] * C)
    q = q_flat.reshape(B, H, W, C).transpose(0, 3, 1, 2)
    h = jax.nn.relu(_deconv2d(q, dw1, db1, 2, 1))
    h = jax.nn.relu(_deconv2d(h, dw2, db2, 2, 1))
    recon = jax.nn.sigmoid(_conv2d(h, w3, b3, 1, 1))
    return (recon, loss, q)


# speedup vs baseline: 1.0230x; 1.0230x over previous
import functools

import jax
import jax.numpy as jnp
from jax.experimental import pallas as pl


def _conv2d(x, w, b, stride, pad):
    out = jax.lax.conv_general_dilated(
        x, w, window_strides=(stride, stride),
        padding=[(pad, pad), (pad, pad)],
        dimension_numbers=('NCHW', 'OIHW', 'NCHW'))
    return out + b[None, :, None, None]


def _deconv2d(x, w, b, stride, pad):
    w_f = jnp.flip(w, axis=(2, 3)).transpose(1, 0, 2, 3)
    k = w.shape[2]
    p = k - 1 - pad
    out = jax.lax.conv_general_dilated(
        x, w_f, window_strides=(1, 1), padding=[(p, p), (p, p)],
        lhs_dilation=(stride, stride),
        dimension_numbers=('NCHW', 'OIHW', 'NCHW'))
    return out + b[None, :, None, None]


def _vq_kernel(flat_ref, rn_ref, emb_ref, cn_ref, q_ref, sq_ref):
    i = pl.program_id(0)
    f = flat_ref[...]
    e = emb_ref[...]
    fe = jax.lax.dot_general(f, e, (((1,), (1,)), ((), ())),
                             preferred_element_type=jnp.float32)
    d2 = rn_ref[...] - 2.0 * fe + cn_ref[...]
    dist = jnp.sqrt(jnp.maximum(d2, 0.0))
    minv = jnp.min(dist, axis=1, keepdims=True)
    kiota = jax.lax.broadcasted_iota(jnp.int32, dist.shape, 1)
    big = jnp.int32(dist.shape[1])
    idx = jnp.min(jnp.where(dist == minv, kiota, big), axis=1)
    onehot = (idx[:, None] == kiota).astype(jnp.float32)
    q = jax.lax.dot_general(onehot, e, (((1,), (0,)), ((), ())),
                            preferred_element_type=jnp.float32)
    q_ref[...] = q
    diff = q - f
    part = jnp.sum(diff * diff).reshape(1, 1)

    @pl.when(i == 0)
    def _init():
        sq_ref[...] = part

    @pl.when(i != 0)
    def _acc():
        sq_ref[...] += part


@functools.partial(jax.jit, static_argnames=("blk",))
def _vq(flat, emb, blk=512):
    n, c = flat.shape
    k = emb.shape[0]
    rn = jnp.sum(flat ** 2, axis=1, keepdims=True)
    cn = jnp.sum(emb ** 2, axis=1)[None, :]
    grid = n // blk
    q, sq = pl.pallas_call(
        _vq_kernel,
        grid=(grid,),
        in_specs=[
            pl.BlockSpec((blk, c), lambda i: (i, 0)),
            pl.BlockSpec((blk, 1), lambda i: (i, 0)),
            pl.BlockSpec((k, c), lambda i: (0, 0)),
            pl.BlockSpec((1, k), lambda i: (0, 0)),
        ],
        out_specs=[
            pl.BlockSpec((blk, c), lambda i: (i, 0)),
            pl.BlockSpec((1, 1), lambda i: (0, 0)),
        ],
        out_shape=[
            jax.ShapeDtypeStruct((n, c), jnp.float32),
            jax.ShapeDtypeStruct((1, 1), jnp.float32),
        ],
    )(flat, rn, emb, cn)
    return q, sq[0, 0]


def kernel(x, w1, b1, w2, b2, emb, dw1, db1, dw2, db2, w3, b3):
    z = jax.nn.relu(_conv2d(x, w1, b1, 2, 1))
    z = jax.nn.relu(_conv2d(z, w2, b2, 2, 1))
    B, C, H, W = z.shape
    flat = z.transpose(0, 2, 3, 1).reshape(-1, C)
    q_flat, sq = _vq(flat, emb)
    loss = 1.25 * sq / (flat.shape[0] * C)
    q = q_flat.reshape(B, H, W, C).transpose(0, 3, 1, 2)
    h = jax.nn.relu(_deconv2d(q, dw1, db1, 2, 1))
    h = jax.nn.relu(_deconv2d(h, dw2, db2, 2, 1))
    recon = jax.nn.sigmoid(_conv2d(h, w3, b3, 1, 1))
    return (recon, loss, q)
